# baseline (speedup 1.0000x reference)
import functools

import jax
import jax.numpy as jnp
from jax import lax
from jax.experimental import pallas as pl
from jax.experimental.pallas import tpu as pltpu
from jax.experimental.pallas import tpu_sc as plsc

N = 10000
NT = 10112
EP = 331776
K = 64
ROWS_PER_SUB = NT // 16

_GATHER_DNUMS = lax.GatherDimensionNumbers(
    offset_dims=(), collapsed_slice_dims=(0,), start_index_map=(0,))


def _perm(v, idx):
    return lax.gather(v, idx[:, None], _GATHER_DNUMS, (1,),
                      mode=lax.GatherScatterMode.PROMISE_IN_BOUNDS)


def _tc_prologue(x2d, W_in, b_in, gamma, beta, Wl0, Wr0):
    BN = 1000

    def body(x_ref, wi_ref, bi_ref, g_ref, be_ref, wl_ref, wr_ref,
             xl_ref, xr_ref):
        h = jnp.dot(x_ref[...], wi_ref[...],
                    preferred_element_type=jnp.float32) + bi_ref[...]
        mu = jnp.mean(h, axis=-1, keepdims=True)
        var = jnp.mean((h - mu) ** 2, axis=-1, keepdims=True)
        h = (h - mu) / jnp.sqrt(var + 1e-5) * g_ref[...] + be_ref[...]
        h = jnp.maximum(h, 0.0)
        xl = jnp.dot(h, wl_ref[...], preferred_element_type=jnp.float32)
        xr = jnp.dot(h, wr_ref[...], preferred_element_type=jnp.float32)
        xl_ref[0] = xl[:, :128]
        xl_ref[1] = xl[:, 128:]
        xr_ref[0] = xr[:, :128]
        xr_ref[1] = xr[:, 128:]

    full = lambda s: pl.BlockSpec(s, lambda i: tuple(0 for _ in s))
    return pl.pallas_call(
        body,
        grid=(N // BN,),
        in_specs=[
            pl.BlockSpec((BN, 128), lambda i: (i, 0)),
            full((128, 64)), full((1, 64)), full((1, 64)), full((1, 64)),
            full((64, 256)), full((64, 256)),
        ],
        out_specs=[
            pl.BlockSpec((2, BN, 128), lambda i: (0, i, 0)),
            pl.BlockSpec((2, BN, 128), lambda i: (0, i, 0)),
        ],
        out_shape=[
            jax.ShapeDtypeStruct((2, N, 128), jnp.float32),
            jax.ShapeDtypeStruct((2, N, 128), jnp.float32),
        ],
    )(x2d, W_in, b_in.reshape(1, 64), gamma.reshape(1, 64),
      beta.reshape(1, 64), Wl0, Wr0)


def _sc_edges_l0(xl_tab, xr_tab, src2, dstg2, dst_eff, att_tab, zrows,
                 zden):
    W = EP // 16
    NB = W // K
    mesh = plsc.VectorSubcoreMesh(core_axis_name="c", subcore_axis_name="s")

    @functools.partial(
        pl.kernel, mesh=mesh,
        out_type=[jax.ShapeDtypeStruct((2, NT, 128), jnp.float32),
                  jax.ShapeDtypeStruct((2, 16, 2 * NT), jnp.float32)],
        scratch_types=[
            pltpu.VMEM((K,), jnp.int32),
            pltpu.VMEM((K,), jnp.int32),
            pltpu.VMEM((K,), jnp.int32),
            pltpu.VMEM((K, 128), jnp.float32),
            pltpu.VMEM((K, 128), jnp.float32),
            pltpu.VMEM((2 * NT + 16,), jnp.float32),
            pltpu.VMEM((128,), jnp.float32),
            pltpu.VMEM_SHARED((NT, 128), jnp.float32),
            pltpu.SemaphoreType.DMA,
            pltpu.SemaphoreType.DMA,
        ],
    )
    def k(xl_hbm, xr_hbm, src_hbm, dstg_hbm, dsts_hbm, att_hbm, z_hbm,
          zd_hbm, out_hbm, den_hbm, idx_s, idx_g, idx_c, xl_v, xr_v,
          den_l, att_v, acc, sem1, sem2):
        c = lax.axis_index("c")
        s = lax.axis_index("s")
        pltpu.sync_copy(z_hbm, acc.at[pl.ds(s * ROWS_PER_SUB, ROWS_PER_SUB)])
        pltpu.sync_copy(zd_hbm, den_l.at[pl.ds(0, 2 * NT)])
        pltpu.sync_copy(att_hbm.at[c], att_v)
        plsc.subcore_barrier()

        att_r = [att_v[pl.ds(j * 16, 16)] for j in range(8)]
        iota = lax.iota(jnp.int32, 16)
        p8 = jnp.bitwise_xor(iota, 8)
        p4 = jnp.bitwise_xor(iota, 4)
        p2 = jnp.bitwise_xor(iota, 2)
        p1 = jnp.bitwise_xor(iota, 1)
        bc0 = jnp.zeros((16,), jnp.int32)
        bc8 = jnp.full((16,), 8, jnp.int32)
        lo8 = iota < 8
        m0 = iota == 0
        m1 = iota == 1
        zero16 = jnp.zeros((16,), jnp.float32)

        def batch(g, carry):
            base = s * W + g * K
            pltpu.sync_copy(src_hbm.at[c, pl.ds(base, K)], idx_s)
            pltpu.sync_copy(dstg_hbm.at[c, pl.ds(base, K)], idx_g)
            pltpu.sync_copy(dsts_hbm.at[pl.ds(base, K)], idx_c)
            cp1 = pltpu.async_copy(xl_hbm.at[idx_s], xl_v, sem1)
            cp2 = pltpu.async_copy(xr_hbm.at[idx_g], xr_v, sem2)
            cp1.wait()
            cp2.wait()

            def group(g2, gcarry):
                gbase = g2 * 16
                idxv = idx_c[pl.ds(gbase, 16)]
                for i in range(16):
                    e = gbase + i
                    xs = [xl_v[e, pl.ds(j * 16, 16)] for j in range(8)]
                    sA = jnp.zeros((16,), jnp.float32)
                    sB = jnp.zeros((16,), jnp.float32)
                    for j in range(8):
                        mj = xs[j] + xr_v[e, pl.ds(j * 16, 16)]
                        mj = jnp.maximum(mj, 0.2 * mj)
                        tj = mj * att_r[j]
                        if j < 4:
                            sA = sA + tj
                        else:
                            sB = sB + tj
                    u = jnp.where(lo8, sA + _perm(sA, p8), sB + _perm(sB, p8))
                    u = u + _perm(u, p4)
                    u = u + _perm(u, p2)
                    u = u + _perm(u, p1)
                    ex = jnp.exp(u)
                    exA = _perm(ex, bc0)
                    exB = _perm(ex, bc8)
                    for j in range(4):
                        xl_v[e, pl.ds(j * 16, 16)] = exA * xs[j]
                    for j in range(4, 8):
                        xl_v[e, pl.ds(j * 16, 16)] = exB * xs[j]
                    d2 = idxv[i] * 2
                    dvec = jnp.where(m0, exA, jnp.where(m1, exB, zero16))
                    den_l[pl.ds(d2, 16)] = den_l[pl.ds(d2, 16)] + dvec
                return gcarry

            lax.fori_loop(0, K // 16, group, 0)
            pltpu.sync_copy(xl_v, acc.at[idx_c], add=True)
            return carry

        lax.fori_loop(0, NB, batch, 0)
        plsc.subcore_barrier()
        pltpu.sync_copy(acc.at[pl.ds(s * ROWS_PER_SUB, ROWS_PER_SUB)],
                        out_hbm.at[c, pl.ds(s * ROWS_PER_SUB, ROWS_PER_SUB)])
        pltpu.sync_copy(den_l.at[pl.ds(0, 2 * NT)], den_hbm.at[c, s])

    return k(xl_tab, xr_tab, src2, dstg2, dst_eff, att_tab, zrows, zden)


def _tc_mid(msg0, den0, bias0, Wl1, Wr1):
    BN = 1000

    def body(a_ref, d_ref, b_ref, wl_ref, wr_ref, xl_ref, xr_ref):
        den = jnp.sum(d_ref[...], axis=1)
        cols = []
        for p in range(2):
            msg = a_ref[p]
            d0 = den[p, :, 0:1] + 1e-16
            d1 = den[p, :, 1:2] + 1e-16
            cols.append(msg[:, 0:64] / d0)
            cols.append(msg[:, 64:128] / d1)
        h1 = jnp.concatenate(cols, axis=1) + b_ref[...]
        h1 = jnp.maximum(h1, 0.0)
        xl_ref[...] = jnp.dot(h1, wl_ref[...],
                              preferred_element_type=jnp.float32)
        xr_ref[...] = jnp.dot(h1, wr_ref[...],
                              preferred_element_type=jnp.float32)

    full = lambda s: pl.BlockSpec(s, lambda i: tuple(0 for _ in s))
    return pl.pallas_call(
        body,
        grid=(N // BN,),
        in_specs=[
            pl.BlockSpec((2, BN, 128), lambda i: (0, i, 0)),
            pl.BlockSpec((2, 16, BN, 2), lambda i: (0, 0, i, 0)),
            full((1, 256)), full((256, 128)), full((256, 128)),
        ],
        out_specs=[
            pl.BlockSpec((BN, 128), lambda i: (i, 0)),
            pl.BlockSpec((BN, 128), lambda i: (i, 0)),
        ],
        out_shape=[
            jax.ShapeDtypeStruct((N, 128), jnp.float32),
            jax.ShapeDtypeStruct((N, 128), jnp.float32),
        ],
    )(msg0, den0, bias0.reshape(1, 256), Wl1, Wr1)


def _sc_edges_l1(xl_tab, xr_tab, src, dstg, dst_eff, att1, zrows, zden):
    W = EP // 32
    NB = W // K
    HALF = EP // 2
    mesh = plsc.VectorSubcoreMesh(core_axis_name="c", subcore_axis_name="s")

    @functools.partial(
        pl.kernel, mesh=mesh,
        out_type=[jax.ShapeDtypeStruct((2, NT, 128), jnp.float32),
                  jax.ShapeDtypeStruct((2, 16, NT), jnp.float32)],
        scratch_types=[
            pltpu.VMEM((K,), jnp.int32),
            pltpu.VMEM((K,), jnp.int32),
            pltpu.VMEM((K,), jnp.int32),
            pltpu.VMEM((K, 128), jnp.float32),
            pltpu.VMEM((K, 128), jnp.float32),
            pltpu.VMEM((NT + 16,), jnp.float32),
            pltpu.VMEM((128,), jnp.float32),
            pltpu.VMEM_SHARED((NT, 128), jnp.float32),
            pltpu.SemaphoreType.DMA,
            pltpu.SemaphoreType.DMA,
        ],
    )
    def k(xl_hbm, xr_hbm, src_hbm, dstg_hbm, dsts_hbm, att_hbm, z_hbm,
          zd_hbm, out_hbm, den_hbm, idx_s, idx_g, idx_c, xl_v, xr_v,
          den_l, att_v, acc, sem1, sem2):
        c = lax.axis_index("c")
        s = lax.axis_index("s")
        pltpu.sync_copy(z_hbm, acc.at[pl.ds(s * ROWS_PER_SUB, ROWS_PER_SUB)])
        pltpu.sync_copy(zd_hbm.at[pl.ds(0, NT)], den_l.at[pl.ds(0, NT)])
        pltpu.sync_copy(att_hbm, att_v)
        plsc.subcore_barrier()

        att_r = [att_v[pl.ds(j * 16, 16)] for j in range(8)]
        iota = lax.iota(jnp.int32, 16)
        p8 = jnp.bitwise_xor(iota, 8)
        p4 = jnp.bitwise_xor(iota, 4)
        p2 = jnp.bitwise_xor(iota, 2)
        p1 = jnp.bitwise_xor(iota, 1)
        m0 = iota == 0
        zero16 = jnp.zeros((16,), jnp.float32)

        def batch(g, carry):
            base = c * HALF + s * W + g * K
            pltpu.sync_copy(src_hbm.at[pl.ds(base, K)], idx_s)
            pltpu.sync_copy(dstg_hbm.at[pl.ds(base, K)], idx_g)
            pltpu.sync_copy(dsts_hbm.at[pl.ds(base, K)], idx_c)
            cp1 = pltpu.async_copy(xl_hbm.at[idx_s], xl_v, sem1)
            cp2 = pltpu.async_copy(xr_hbm.at[idx_g], xr_v, sem2)
            cp1.wait()
            cp2.wait()

            def group(g2, gcarry):
                gbase = g2 * 16
                idxv = idx_c[pl.ds(gbase, 16)]
                for i in range(16):
                    e = gbase + i
                    xs = [xl_v[e, pl.ds(j * 16, 16)] for j in range(8)]
                    sA = jnp.zeros((16,), jnp.float32)
                    for j in range(8):
                        mj = xs[j] + xr_v[e, pl.ds(j * 16, 16)]
                        mj = jnp.maximum(mj, 0.2 * mj)
                        sA = sA + mj * att_r[j]
                    u = sA + _perm(sA, p8)
                    u = u + _perm(u, p4)
                    u = u + _perm(u, p2)
                    u = u + _perm(u, p1)
                    ex = jnp.exp(u)
                    for j in range(8):
                        xl_v[e, pl.ds(j * 16, 16)] = ex * xs[j]
                    d = idxv[i]
                    dvec = jnp.where(m0, ex, zero16)
                    den_l[pl.ds(d, 16)] = den_l[pl.ds(d, 16)] + dvec
                return gcarry

            lax.fori_loop(0, K // 16, group, 0)
            pltpu.sync_copy(xl_v, acc.at[idx_c], add=True)
            return carry

        lax.fori_loop(0, NB, batch, 0)
        plsc.subcore_barrier()
        pltpu.sync_copy(acc.at[pl.ds(s * ROWS_PER_SUB, ROWS_PER_SUB)],
                        out_hbm.at[c, pl.ds(s * ROWS_PER_SUB, ROWS_PER_SUB)])
        pltpu.sync_copy(den_l.at[pl.ds(0, NT)], den_hbm.at[c, s])

    return k(xl_tab, xr_tab, src, dstg, dst_eff, att1, zrows, zden)


def _tc_epilogue(msg1, den1, bias1):
    BN = 1000

    def body(a_ref, d_ref, b_ref, o_ref):
        msg = a_ref[0] + a_ref[1]
        den = jnp.sum(d_ref[...], axis=(0, 1)) + 1e-16
        o_ref[...] = jnp.maximum(msg / den + b_ref[...], 0.0)

    full = lambda s: pl.BlockSpec(s, lambda i: tuple(0 for _ in s))
    return pl.pallas_call(
        body,
        grid=(N // BN,),
        in_specs=[
            pl.BlockSpec((2, BN, 128), lambda i: (0, i, 0)),
            pl.BlockSpec((2, 16, BN, 1), lambda i: (0, 0, i, 0)),
            full((1, 128)),
        ],
        out_specs=pl.BlockSpec((BN, 128), lambda i: (i, 0)),
        out_shape=jax.ShapeDtypeStruct((N, 128), jnp.float32),
    )(msg1, den1, bias1.reshape(1, 128))


def kernel(x, edge_index, W_in, b_in, gamma, beta, Wl0, Wr0, att0, bias0,
           Wl1, Wr1, att1, bias1):
    e_src = edge_index[0]
    e_dst = edge_index[1]
    mask = e_src != e_dst
    loops = jnp.arange(N, dtype=jnp.int32)
    npad = EP - (e_src.shape[0] + N)
    src = jnp.concatenate([e_src, loops, jnp.zeros((npad,), jnp.int32)])
    dst = jnp.concatenate([e_dst, loops, jnp.zeros((npad,), jnp.int32)])
    emask = jnp.concatenate([mask, jnp.ones((N,), bool),
                             jnp.zeros((npad,), bool)])
    dst_eff = jnp.where(emask, dst, N).astype(jnp.int32)
    dstg = jnp.where(emask, dst, 0).astype(jnp.int32)
    src2 = jnp.stack([src, src + N])
    dstg2 = jnp.stack([dstg, dstg + N])
    att_tab = att0.reshape(2, 128)
    zrows = jnp.zeros((ROWS_PER_SUB, 128), jnp.float32)
    zden = jnp.zeros((2 * NT,), jnp.float32)

    xl0, xr0 = _tc_prologue(x.reshape(N, 128), W_in, b_in, gamma, beta,
                            Wl0, Wr0)
    msg0, den0 = _sc_edges_l0(xl0.reshape(2 * N, 128), xr0.reshape(2 * N, 128),
                              src2, dstg2, dst_eff, att_tab, zrows, zden)
    xl1, xr1 = _tc_mid(msg0, den0.reshape(2, 16, NT, 2), bias0, Wl1, Wr1)
    msg1, den1 = _sc_edges_l1(xl1, xr1, src, dstg, dst_eff,
                              att1.reshape(128), zrows, zden)
    out = _tc_epilogue(msg1, den1.reshape(2, 16, NT, 1), bias1)
    return out.reshape(1, N, 128)


# speedup vs baseline: 1.1019x; 1.1019x over previous
import functools

import jax
import jax.numpy as jnp
from jax import lax
from jax.experimental import pallas as pl
from jax.experimental.pallas import tpu as pltpu
from jax.experimental.pallas import tpu_sc as plsc

N = 10000
NT = 10112
EP = 331776
K = 96
ROWS_PER_SUB = NT // 16

_GATHER_DNUMS = lax.GatherDimensionNumbers(
    offset_dims=(), collapsed_slice_dims=(0,), start_index_map=(0,))


def _perm(v, idx):
    return lax.gather(v, idx[:, None], _GATHER_DNUMS, (1,),
                      mode=lax.GatherScatterMode.PROMISE_IN_BOUNDS)


def _tc_prologue(x2d, W_in, b_in, gamma, beta, Wl0, Wr0):
    BN = 1000

    def body(x_ref, wi_ref, bi_ref, g_ref, be_ref, wl_ref, wr_ref,
             xl_ref, xr_ref):
        h = jnp.dot(x_ref[...], wi_ref[...],
                    preferred_element_type=jnp.float32) + bi_ref[...]
        mu = jnp.mean(h, axis=-1, keepdims=True)
        var = jnp.mean((h - mu) ** 2, axis=-1, keepdims=True)
        h = (h - mu) / jnp.sqrt(var + 1e-5) * g_ref[...] + be_ref[...]
        h = jnp.maximum(h, 0.0)
        xl = jnp.dot(h, wl_ref[...], preferred_element_type=jnp.float32)
        xr = jnp.dot(h, wr_ref[...], preferred_element_type=jnp.float32)
        xl_ref[0] = xl[:, :128]
        xl_ref[1] = xl[:, 128:]
        xr_ref[0] = xr[:, :128]
        xr_ref[1] = xr[:, 128:]

    full = lambda s: pl.BlockSpec(s, lambda i: tuple(0 for _ in s))
    return pl.pallas_call(
        body,
        grid=(N // BN,),
        in_specs=[
            pl.BlockSpec((BN, 128), lambda i: (i, 0)),
            full((128, 64)), full((1, 64)), full((1, 64)), full((1, 64)),
            full((64, 256)), full((64, 256)),
        ],
        out_specs=[
            pl.BlockSpec((2, BN, 128), lambda i: (0, i, 0)),
            pl.BlockSpec((2, BN, 128), lambda i: (0, i, 0)),
        ],
        out_shape=[
            jax.ShapeDtypeStruct((2, N, 128), jnp.float32),
            jax.ShapeDtypeStruct((2, N, 128), jnp.float32),
        ],
    )(x2d, W_in, b_in.reshape(1, 64), gamma.reshape(1, 64),
      beta.reshape(1, 64), Wl0, Wr0)


def _sc_edges_l0(xl_tab, xr_tab, src2, dstg2, dst_eff, att_tab, zrows,
                 zden):
    W = EP // 16
    NB = W // K
    mesh = plsc.VectorSubcoreMesh(core_axis_name="c", subcore_axis_name="s")

    @functools.partial(
        pl.kernel, mesh=mesh,
        out_type=[jax.ShapeDtypeStruct((2, NT, 128), jnp.float32),
                  jax.ShapeDtypeStruct((2, 16, 2 * NT), jnp.float32)],
        scratch_types=[
            pltpu.VMEM((K,), jnp.int32),
            pltpu.VMEM((K,), jnp.int32),
            pltpu.VMEM((K,), jnp.int32),
            pltpu.VMEM((K, 128), jnp.float32),
            pltpu.VMEM((K, 128), jnp.float32),
            pltpu.VMEM((2 * NT + 16,), jnp.float32),
            pltpu.VMEM((128,), jnp.float32),
            pltpu.VMEM_SHARED((NT, 128), jnp.float32),
            pltpu.SemaphoreType.DMA,
            pltpu.SemaphoreType.DMA,
        ],
    )
    def k(xl_hbm, xr_hbm, src_hbm, dstg_hbm, dsts_hbm, att_hbm, z_hbm,
          zd_hbm, out_hbm, den_hbm, idx_s, idx_g, idx_c, xl_v, xr_v,
          den_l, att_v, acc, sem1, sem2):
        c = lax.axis_index("c")
        s = lax.axis_index("s")
        pltpu.sync_copy(z_hbm, acc.at[pl.ds(s * ROWS_PER_SUB, ROWS_PER_SUB)])
        pltpu.sync_copy(zd_hbm, den_l.at[pl.ds(0, 2 * NT)])
        pltpu.sync_copy(att_hbm.at[c], att_v)
        plsc.subcore_barrier()

        att_r = [att_v[pl.ds(j * 16, 16)] for j in range(8)]
        iota = lax.iota(jnp.int32, 16)
        p8 = jnp.bitwise_xor(iota, 8)
        p4 = jnp.bitwise_xor(iota, 4)
        p2 = jnp.bitwise_xor(iota, 2)
        p1 = jnp.bitwise_xor(iota, 1)
        bc0 = jnp.zeros((16,), jnp.int32)
        bc8 = jnp.full((16,), 8, jnp.int32)
        lo8 = iota < 8
        m0 = iota == 0
        m1 = iota == 1
        zero16 = jnp.zeros((16,), jnp.float32)

        def batch(g, carry):
            base = s * W + g * K
            pltpu.sync_copy(src_hbm.at[pl.ds(c * EP + base, K)], idx_s)
            pltpu.sync_copy(dstg_hbm.at[pl.ds(c * EP + base, K)], idx_g)
            pltpu.sync_copy(dsts_hbm.at[pl.ds(base, K)], idx_c)
            cp1 = pltpu.async_copy(xl_hbm.at[idx_s], xl_v, sem1)
            cp2 = pltpu.async_copy(xr_hbm.at[idx_g], xr_v, sem2)
            cp1.wait()
            cp2.wait()

            def group(g2, gcarry):
                gbase = g2 * 16
                idxv = idx_c[pl.ds(gbase, 16)]
                for i in range(16):
                    e = gbase + i
                    xs = [xl_v[e, pl.ds(j * 16, 16)] for j in range(8)]
                    sA = jnp.zeros((16,), jnp.float32)
                    sB = jnp.zeros((16,), jnp.float32)
                    for j in range(8):
                        mj = xs[j] + xr_v[e, pl.ds(j * 16, 16)]
                        mj = jnp.maximum(mj, 0.2 * mj)
                        tj = mj * att_r[j]
                        if j < 4:
                            sA = sA + tj
                        else:
                            sB = sB + tj
                    u = jnp.where(lo8, sA + _perm(sA, p8), sB + _perm(sB, p8))
                    u = u + _perm(u, p4)
                    u = u + _perm(u, p2)
                    u = u + _perm(u, p1)
                    ex = jnp.exp(u)
                    exA = _perm(ex, bc0)
                    exB = _perm(ex, bc8)
                    for j in range(4):
                        xl_v[e, pl.ds(j * 16, 16)] = exA * xs[j]
                    for j in range(4, 8):
                        xl_v[e, pl.ds(j * 16, 16)] = exB * xs[j]
                    d2 = idxv[i] * 2
                    dvec = jnp.where(m0, exA, jnp.where(m1, exB, zero16))
                    den_l[pl.ds(d2, 16)] = den_l[pl.ds(d2, 16)] + dvec
                return gcarry

            lax.fori_loop(0, K // 16, group, 0)
            pltpu.sync_copy(xl_v, acc.at[idx_c], add=True)
            return carry

        lax.fori_loop(0, NB, batch, 0)
        plsc.subcore_barrier()
        pltpu.sync_copy(acc.at[pl.ds(s * ROWS_PER_SUB, ROWS_PER_SUB)],
                        out_hbm.at[c, pl.ds(s * ROWS_PER_SUB, ROWS_PER_SUB)])
        pltpu.sync_copy(den_l.at[pl.ds(0, 2 * NT)], den_hbm.at[c, s])

    return k(xl_tab, xr_tab, src2, dstg2, dst_eff, att_tab, zrows, zden)


def _tc_mid(msg0, den0, bias0, Wl1, Wr1):
    BN = 1000

    def body(a_ref, d_ref, b_ref, wl_ref, wr_ref, xl_ref, xr_ref):
        den = jnp.sum(d_ref[...], axis=1)
        cols = []
        for p in range(2):
            msg = a_ref[p]
            d0 = den[p, :, 0:1] + 1e-16
            d1 = den[p, :, 1:2] + 1e-16
            cols.append(msg[:, 0:64] / d0)
            cols.append(msg[:, 64:128] / d1)
        h1 = jnp.concatenate(cols, axis=1) + b_ref[...]
        h1 = jnp.maximum(h1, 0.0)
        xl_ref[...] = jnp.dot(h1, wl_ref[...],
                              preferred_element_type=jnp.float32)
        xr_ref[...] = jnp.dot(h1, wr_ref[...],
                              preferred_element_type=jnp.float32)

    full = lambda s: pl.BlockSpec(s, lambda i: tuple(0 for _ in s))
    return pl.pallas_call(
        body,
        grid=(N // BN,),
        in_specs=[
            pl.BlockSpec((2, BN, 128), lambda i: (0, i, 0)),
            pl.BlockSpec((2, 16, BN, 2), lambda i: (0, 0, i, 0)),
            full((1, 256)), full((256, 128)), full((256, 128)),
        ],
        out_specs=[
            pl.BlockSpec((BN, 128), lambda i: (i, 0)),
            pl.BlockSpec((BN, 128), lambda i: (i, 0)),
        ],
        out_shape=[
            jax.ShapeDtypeStruct((N, 128), jnp.float32),
            jax.ShapeDtypeStruct((N, 128), jnp.float32),
        ],
    )(msg0, den0, bias0.reshape(1, 256), Wl1, Wr1)


def _sc_edges_l1(xl_tab, xr_tab, src, dstg, dst_eff, att1, zrows, zden):
    W = EP // 32
    NB = W // K
    HALF = EP // 2
    mesh = plsc.VectorSubcoreMesh(core_axis_name="c", subcore_axis_name="s")

    @functools.partial(
        pl.kernel, mesh=mesh,
        out_type=[jax.ShapeDtypeStruct((2, NT, 128), jnp.float32),
                  jax.ShapeDtypeStruct((2, 16, NT), jnp.float32)],
        scratch_types=[
            pltpu.VMEM((K,), jnp.int32),
            pltpu.VMEM((K,), jnp.int32),
            pltpu.VMEM((K,), jnp.int32),
            pltpu.VMEM((K, 128), jnp.float32),
            pltpu.VMEM((K, 128), jnp.float32),
            pltpu.VMEM((NT + 16,), jnp.float32),
            pltpu.VMEM((128,), jnp.float32),
            pltpu.VMEM_SHARED((NT, 128), jnp.float32),
            pltpu.SemaphoreType.DMA,
            pltpu.SemaphoreType.DMA,
        ],
    )
    def k(xl_hbm, xr_hbm, src_hbm, dstg_hbm, dsts_hbm, att_hbm, z_hbm,
          zd_hbm, out_hbm, den_hbm, idx_s, idx_g, idx_c, xl_v, xr_v,
          den_l, att_v, acc, sem1, sem2):
        c = lax.axis_index("c")
        s = lax.axis_index("s")
        pltpu.sync_copy(z_hbm, acc.at[pl.ds(s * ROWS_PER_SUB, ROWS_PER_SUB)])
        pltpu.sync_copy(zd_hbm.at[pl.ds(0, NT)], den_l.at[pl.ds(0, NT)])
        pltpu.sync_copy(att_hbm, att_v)
        plsc.subcore_barrier()

        att_r = [att_v[pl.ds(j * 16, 16)] for j in range(8)]
        iota = lax.iota(jnp.int32, 16)
        p8 = jnp.bitwise_xor(iota, 8)
        p4 = jnp.bitwise_xor(iota, 4)
        p2 = jnp.bitwise_xor(iota, 2)
        p1 = jnp.bitwise_xor(iota, 1)
        m0 = iota == 0
        zero16 = jnp.zeros((16,), jnp.float32)

        def batch(g, carry):
            base = c * HALF + s * W + g * K
            pltpu.sync_copy(src_hbm.at[pl.ds(base, K)], idx_s)
            pltpu.sync_copy(dstg_hbm.at[pl.ds(base, K)], idx_g)
            pltpu.sync_copy(dsts_hbm.at[pl.ds(base, K)], idx_c)
            cp1 = pltpu.async_copy(xl_hbm.at[idx_s], xl_v, sem1)
            cp2 = pltpu.async_copy(xr_hbm.at[idx_g], xr_v, sem2)
            cp1.wait()
            cp2.wait()

            def group(g2, gcarry):
                gbase = g2 * 16
                idxv = idx_c[pl.ds(gbase, 16)]
                for i in range(16):
                    e = gbase + i
                    xs = [xl_v[e, pl.ds(j * 16, 16)] for j in range(8)]
                    sA = jnp.zeros((16,), jnp.float32)
                    for j in range(8):
                        mj = xs[j] + xr_v[e, pl.ds(j * 16, 16)]
                        mj = jnp.maximum(mj, 0.2 * mj)
                        sA = sA + mj * att_r[j]
                    u = sA + _perm(sA, p8)
                    u = u + _perm(u, p4)
                    u = u + _perm(u, p2)
                    u = u + _perm(u, p1)
                    ex = jnp.exp(u)
                    for j in range(8):
                        xl_v[e, pl.ds(j * 16, 16)] = ex * xs[j]
                    d = idxv[i]
                    dvec = jnp.where(m0, ex, zero16)
                    den_l[pl.ds(d, 16)] = den_l[pl.ds(d, 16)] + dvec
                return gcarry

            lax.fori_loop(0, K // 16, group, 0)
            pltpu.sync_copy(xl_v, acc.at[idx_c], add=True)
            return carry

        lax.fori_loop(0, NB, batch, 0)
        plsc.subcore_barrier()
        pltpu.sync_copy(acc.at[pl.ds(s * ROWS_PER_SUB, ROWS_PER_SUB)],
                        out_hbm.at[c, pl.ds(s * ROWS_PER_SUB, ROWS_PER_SUB)])
        pltpu.sync_copy(den_l.at[pl.ds(0, NT)], den_hbm.at[c, s])

    return k(xl_tab, xr_tab, src, dstg, dst_eff, att1, zrows, zden)


def _tc_epilogue(msg1, den1, bias1):
    BN = 1000

    def body(a_ref, d_ref, b_ref, o_ref):
        msg = a_ref[0] + a_ref[1]
        den = jnp.sum(d_ref[...], axis=(0, 1)) + 1e-16
        o_ref[...] = jnp.maximum(msg / den + b_ref[...], 0.0)

    full = lambda s: pl.BlockSpec(s, lambda i: tuple(0 for _ in s))
    return pl.pallas_call(
        body,
        grid=(N // BN,),
        in_specs=[
            pl.BlockSpec((2, BN, 128), lambda i: (0, i, 0)),
            pl.BlockSpec((2, 16, BN, 1), lambda i: (0, 0, i, 0)),
            full((1, 128)),
        ],
        out_specs=pl.BlockSpec((BN, 128), lambda i: (i, 0)),
        out_shape=jax.ShapeDtypeStruct((N, 128), jnp.float32),
    )(msg1, den1, bias1.reshape(1, 128))


def kernel(x, edge_index, W_in, b_in, gamma, beta, Wl0, Wr0, att0, bias0,
           Wl1, Wr1, att1, bias1):
    e_src = edge_index[0]
    e_dst = edge_index[1]
    mask = e_src != e_dst
    loops = jnp.arange(N, dtype=jnp.int32)
    npad = EP - (e_src.shape[0] + N)
    src = jnp.concatenate([e_src, loops, jnp.zeros((npad,), jnp.int32)])
    dst = jnp.concatenate([e_dst, loops, jnp.zeros((npad,), jnp.int32)])
    emask = jnp.concatenate([mask, jnp.ones((N,), bool),
                             jnp.zeros((npad,), bool)])
    dst_eff = jnp.where(emask, dst, N).astype(jnp.int32)
    dstg = jnp.where(emask, dst, 0).astype(jnp.int32)
    src2 = jnp.concatenate([src, src + N])
    dstg2 = jnp.concatenate([dstg, dstg + N])
    att_tab = att0.reshape(2, 128)
    zrows = jnp.zeros((ROWS_PER_SUB, 128), jnp.float32)
    zden = jnp.zeros((2 * NT,), jnp.float32)

    xl0, xr0 = _tc_prologue(x.reshape(N, 128), W_in, b_in, gamma, beta,
                            Wl0, Wr0)
    msg0, den0 = _sc_edges_l0(xl0.reshape(2 * N, 128), xr0.reshape(2 * N, 128),
                              src2, dstg2, dst_eff, att_tab, zrows, zden)
    xl1, xr1 = _tc_mid(msg0, den0.reshape(2, 16, NT, 2), bias0, Wl1, Wr1)
    msg1, den1 = _sc_edges_l1(xl1, xr1, src, dstg, dst_eff,
                              att1.reshape(128), zrows, zden)
    out = _tc_epilogue(msg1, den1.reshape(2, 16, NT, 1), bias1)
    return out.reshape(1, N, 128)
